# Initial kernel scaffold; baseline (speedup 1.0000x reference)
import jax
import jax.numpy as jnp
from jax.experimental import pallas as pl


def kernel(x, enc1_w, enc1_b, enc2_w, enc2_b, dec1_w, dec1_b, dec2_w, dec2_b, proto_key_data):
    raise NotImplementedError("write your pallas kernel here")


# speedup vs baseline: 1.2682x; 1.2682x over previous
import functools

import jax
import jax.numpy as jnp
from jax import lax
from jax.experimental import pallas as pl
from jax.experimental.pallas import tpu as pltpu

_EPS = 1e-12
_N_CLUSTERS = 16
_DUR = 3


def _unit_rows(v):
    n = jnp.sqrt(jnp.sum(v * v, axis=-1, keepdims=True))
    return v / jnp.maximum(n, _EPS)


def _wsum(a, cout):
    hh = a.shape[0]
    zcol = jnp.zeros((hh, 1, cout), a.dtype)
    left = jnp.concatenate([zcol, a[:, :-1, 0:cout]], axis=1)
    right = jnp.concatenate([a[:, 1:, 2 * cout:3 * cout], zcol], axis=1)
    return a[:, :, cout:2 * cout] + left + right


def _eye(m, dtype):
    return (lax.broadcasted_iota(jnp.int32, (m, m), 0) ==
            lax.broadcasted_iota(jnp.int32, (m, m), 1)).astype(dtype)


def _store3(ref, val, hh, ww, c):
    zr = jnp.zeros((2, ww, 3 * c), ref.dtype)
    ref[0:2] = zr
    ref[hh:hh + 2] = zr
    for b_ in range(3):
        ref[b_:b_ + hh, :, b_ * c:(b_ + 1) * c] = val


def _fused_body(xp_ref, w1_ref, b1_ref, w2_ref, b2_ref, w3_ref, b3_ref,
                w4_ref, b4_ref, bb_ref, p0_ref,
                xbar_ref, znc_ref, mask_ref, logits_ref, idx_ref,
                h1p_ref, zp_ref, d1p_ref,
                *, hh, ww, n_pix, n_iters, sk_iters, inv_eps):
    n = hh * ww
    cin = xp_ref.shape[-1]
    hid = w2_ref.shape[0] // 3
    nz = w2_ref.shape[1] // 3
    k = p0_ref.shape[1]

    xpv = xp_ref[0]
    big = jnp.concatenate(
        [xpv[di:di + hh].reshape(n, cin) for di in range(3)], axis=1)
    a1 = jnp.dot(big, w1_ref[...], preferred_element_type=jnp.float32)
    h1 = _wsum(a1.reshape(hh, ww, 3 * hid), hid)
    h1 = jnp.maximum(h1 + b1_ref[...].reshape(1, 1, hid), 0.0)

    _store3(h1p_ref, h1, hh, ww, hid)
    a2 = jnp.dot(h1p_ref[1:hh + 1].reshape(n, 3 * hid), w2_ref[...],
                 preferred_element_type=jnp.float32)
    z3 = _wsum(a2.reshape(hh, ww, 3 * nz), nz) + b2_ref[...].reshape(1, 1, nz)
    z = z3.reshape(n, nz)

    _store3(zp_ref, z3.astype(jnp.bfloat16), hh, ww, nz)
    a3 = jnp.dot(zp_ref[1:hh + 1].reshape(n, 3 * nz), w3_ref[...],
                 preferred_element_type=jnp.float32)
    d1 = _wsum(a3.reshape(hh, ww, 3 * hid), hid)
    d1 = jnp.maximum(d1 + b3_ref[...].reshape(1, 1, hid), 0.0)

    _store3(d1p_ref, d1.astype(jnp.bfloat16), hh, ww, hid)
    a4 = jnp.dot(d1p_ref[1:hh + 1].reshape(n, 3 * hid), w4_ref[...],
                 preferred_element_type=jnp.float32)
    xbar = _wsum(a4.reshape(hh, ww, 3 * cin), cin)
    xbar = (xbar + b4_ref[...].reshape(1, 1, cin)).reshape(n, cin)

    xbar_ref[0] = lax.dot_general(_eye(cin, jnp.float32), xbar,
                                  (((1,), (1,)), ((), ())),
                                  preferred_element_type=jnp.float32)
    znc_ref[0] = lax.dot_general(_eye(nz, jnp.float32), z,
                                 (((1,), (1,)), ((), ())),
                                 preferred_element_type=jnp.float32)

    bb = bb_ref[...]
    proto = lax.dot_general(bb, z, (((1,), (0,)), ((), ())),
                            preferred_element_type=jnp.float32) / n_pix
    proto = _unit_rows(proto)
    zn = _unit_rows(z)
    sim = lax.dot_general(proto, zn, (((1,), (1,)), ((), ())),
                          preferred_element_type=jnp.float32)
    smin = jnp.min(sim, axis=-1, keepdims=True)
    smax = jnp.max(sim, axis=-1, keepdims=True)
    mask_ref[0] = 1.0 - (sim - smin) / jnp.maximum(smax - smin, 1e-12)

    protos = _unit_rows(p0_ref[0])
    row = lax.broadcasted_iota(jnp.int32, (k, n), 0)
    q = jnp.zeros((k, n), jnp.float32)
    idx = jnp.zeros((1, n), jnp.int32)
    for _ in range(n_iters):
        s = lax.dot_general(protos, zn, (((1,), (1,)), ((), ())),
                            preferred_element_type=jnp.float32)
        smx = jnp.max(s, axis=0, keepdims=True)
        e = jnp.exp(s - smx)
        s = e * pl.reciprocal(jnp.sum(e, axis=0, keepdims=True), approx=True)
        p = jnp.exp(s * inv_eps)
        for _ in range(sk_iters):
            p = p * pl.reciprocal(jnp.sum(p, axis=1, keepdims=True),
                                  approx=True)
            p = p * pl.reciprocal(jnp.sum(p, axis=0, keepdims=True),
                                  approx=True)
        q = p
        qmax = jnp.max(q, axis=0, keepdims=True)
        idx = jnp.min(jnp.where(q >= qmax, row, k), axis=0, keepdims=True)
        one_hot = (row == idx).astype(jnp.float32)
        counts = jnp.sum(one_hot, axis=1, keepdims=True)
        new_p = lax.dot_general(one_hot, z, (((1,), (0,)), ((), ())),
                                preferred_element_type=jnp.float32)
        new_p = new_p / jnp.maximum(counts, 1.0)
        protos = _unit_rows(new_p)
    logits_ref[0] = q
    idx_ref[0] = idx


def _wcat(w, dtype):
    nine, cin, cout = w.shape
    assert nine == 9
    w = w.reshape(3, 3, cin, cout).transpose(0, 2, 1, 3)
    return w.reshape(3, cin, 3 * cout).astype(dtype)


def kernel(x, enc1_w, enc1_b, enc2_w, enc2_b, dec1_w, dec1_b,
           dec2_w, dec2_b, proto_key_data):
    b, cin, hh, ww = x.shape
    n = hh * ww
    hid = enc1_w.shape[-1]
    nz = enc2_w.shape[-1]
    k = _N_CLUSTERS

    xph = jnp.pad(jnp.transpose(x, (0, 2, 3, 1)),
                  ((0, 0), (1, 1), (0, 0), (0, 0)))
    w1 = _wcat(enc1_w, jnp.float32).reshape(3 * cin, 3 * hid)
    w2 = _wcat(enc2_w, jnp.float32)[::-1].reshape(3 * hid, 3 * nz)
    w3 = _wcat(dec1_w, jnp.bfloat16)[::-1].reshape(3 * nz, 3 * hid)
    w4 = _wcat(dec2_w, jnp.bfloat16)[::-1].reshape(3 * hid, 3 * cin)

    inner = jnp.zeros((hh - 2 * _DUR, ww - 2 * _DUR), jnp.float32)
    bb = jnp.pad(inner, ((_DUR, _DUR), (_DUR, _DUR)),
                 constant_values=1.0).reshape(1, n)
    n_pix = float(hh * ww - (hh - 2 * _DUR) * (ww - 2 * _DUR))

    proto0 = jax.random.normal(jax.random.wrap_key_data(proto_key_data),
                               (b, k, nz), jnp.float32)

    xbar_t, znc, mask_ln, logits_kn, idx_ln = pl.pallas_call(
        functools.partial(_fused_body, hh=hh, ww=ww, n_pix=n_pix,
                          n_iters=3, sk_iters=3, inv_eps=20.0),
        out_shape=(
            jax.ShapeDtypeStruct((b, cin, n), jnp.float32),
            jax.ShapeDtypeStruct((b, nz, n), jnp.float32),
            jax.ShapeDtypeStruct((b, 1, n), jnp.float32),
            jax.ShapeDtypeStruct((b, k, n), jnp.float32),
            jax.ShapeDtypeStruct((b, 1, n), jnp.int32),
        ),
        grid=(b,),
        in_specs=[
            pl.BlockSpec((1, hh + 2, ww, cin), lambda i: (i, 0, 0, 0)),
            pl.BlockSpec((3 * cin, 3 * hid), lambda i: (0, 0)),
            pl.BlockSpec((1, hid), lambda i: (0, 0)),
            pl.BlockSpec((3 * hid, 3 * nz), lambda i: (0, 0)),
            pl.BlockSpec((1, nz), lambda i: (0, 0)),
            pl.BlockSpec((3 * nz, 3 * hid), lambda i: (0, 0)),
            pl.BlockSpec((1, hid), lambda i: (0, 0)),
            pl.BlockSpec((3 * hid, 3 * cin), lambda i: (0, 0)),
            pl.BlockSpec((1, cin), lambda i: (0, 0)),
            pl.BlockSpec((1, n), lambda i: (0, 0)),
            pl.BlockSpec((1, k, nz), lambda i: (i, 0, 0)),
        ],
        out_specs=(
            pl.BlockSpec((1, cin, n), lambda i: (i, 0, 0)),
            pl.BlockSpec((1, nz, n), lambda i: (i, 0, 0)),
            pl.BlockSpec((1, 1, n), lambda i: (i, 0, 0)),
            pl.BlockSpec((1, k, n), lambda i: (i, 0, 0)),
            pl.BlockSpec((1, 1, n), lambda i: (i, 0, 0)),
        ),
        scratch_shapes=[
            pltpu.VMEM((hh + 2, ww, 3 * hid), jnp.float32),
            pltpu.VMEM((hh + 2, ww, 3 * nz), jnp.bfloat16),
            pltpu.VMEM((hh + 2, ww, 3 * hid), jnp.bfloat16),
        ],
        compiler_params=pltpu.CompilerParams(
            dimension_semantics=("parallel",)),
    )(xph, w1, enc1_b, w2, enc2_b, w3, dec1_b, w4, dec2_b, bb, proto0)

    x_bar = xbar_t.reshape(b, cin, hh, ww)
    z_nchw = znc.reshape(b, nz, hh, ww)
    mask = mask_ln.reshape(b, n, 1)
    logits = logits_kn.reshape(b, k, hh, ww)
    indexes = idx_ln.reshape(b, hh, ww)
    return x_bar, z_nchw, mask, logits, indexes


# speedup vs baseline: 2.0175x; 1.5909x over previous
import functools

import jax
import jax.numpy as jnp
from jax import lax
from jax.experimental import pallas as pl
from jax.experimental.pallas import tpu as pltpu

_EPS = 1e-12
_N_CLUSTERS = 16
_DUR = 3


def _unit_rows(v):
    n = jnp.sqrt(jnp.sum(v * v, axis=-1, keepdims=True))
    return v / jnp.maximum(n, _EPS)


def _wsum(a, cout):
    hh = a.shape[0]
    zcol = jnp.zeros((hh, 1, cout), a.dtype)
    left = jnp.concatenate([zcol, a[:, :-1, 0:cout]], axis=1)
    right = jnp.concatenate([a[:, 1:, 2 * cout:3 * cout], zcol], axis=1)
    return a[:, :, cout:2 * cout] + left + right


def _eye(m, dtype):
    return (lax.broadcasted_iota(jnp.int32, (m, m), 0) ==
            lax.broadcasted_iota(jnp.int32, (m, m), 1)).astype(dtype)


def _store3(ref, val, hh, ww, c):
    zr = jnp.zeros((2, ww, 3 * c), ref.dtype)
    ref[0:2] = zr
    ref[hh:hh + 2] = zr
    for b_ in range(3):
        ref[b_:b_ + hh, :, b_ * c:(b_ + 1) * c] = val


def _fused_body(xp_ref, w1_ref, b1_ref, w2_ref, b2_ref, w3_ref, b3_ref,
                w4_ref, b4_ref, bb_ref, p0_ref,
                xbar_ref, znc_ref, mask_ref, logits_ref, idx_ref,
                h1p_ref, zp_ref, d1p_ref,
                *, hh, ww, n_pix, n_iters, sk_iters, inv_eps):
    n = hh * ww
    cin = xp_ref.shape[1]
    hid = w2_ref.shape[0] // 3
    nz = w2_ref.shape[1] // 3
    k = p0_ref.shape[1]

    xt = jnp.transpose(xp_ref[0], (1, 2, 0))
    zrow = jnp.zeros((1, ww, cin), jnp.float32)
    xpv = jnp.concatenate([zrow, xt, zrow], axis=0)
    big = jnp.concatenate(
        [xpv[di:di + hh].reshape(n, cin) for di in range(3)], axis=1)
    a1 = jnp.dot(big, w1_ref[...], preferred_element_type=jnp.float32)
    h1 = _wsum(a1.reshape(hh, ww, 3 * hid), hid)
    h1 = jnp.maximum(h1 + b1_ref[...].reshape(1, 1, hid), 0.0)

    _store3(h1p_ref, h1, hh, ww, hid)
    a2 = jnp.dot(h1p_ref[1:hh + 1].reshape(n, 3 * hid), w2_ref[...],
                 preferred_element_type=jnp.float32)
    z3 = _wsum(a2.reshape(hh, ww, 3 * nz), nz) + b2_ref[...].reshape(1, 1, nz)
    z = z3.reshape(n, nz)

    _store3(zp_ref, z3.astype(jnp.bfloat16), hh, ww, nz)
    a3 = jnp.dot(zp_ref[1:hh + 1].reshape(n, 3 * nz), w3_ref[...],
                 preferred_element_type=jnp.float32)
    d1 = _wsum(a3.reshape(hh, ww, 3 * hid), hid)
    d1 = jnp.maximum(d1 + b3_ref[...].reshape(1, 1, hid), 0.0)

    _store3(d1p_ref, d1.astype(jnp.bfloat16), hh, ww, hid)
    a4 = jnp.dot(d1p_ref[1:hh + 1].reshape(n, 3 * hid), w4_ref[...],
                 preferred_element_type=jnp.float32)
    xbar = _wsum(a4.reshape(hh, ww, 3 * cin), cin)
    xbar = (xbar + b4_ref[...].reshape(1, 1, cin)).reshape(n, cin)

    xbar_ref[0] = lax.dot_general(_eye(cin, jnp.float32), xbar,
                                  (((1,), (1,)), ((), ())),
                                  preferred_element_type=jnp.float32)
    znc_ref[0] = lax.dot_general(_eye(nz, jnp.float32), z,
                                 (((1,), (1,)), ((), ())),
                                 preferred_element_type=jnp.float32)

    bb = bb_ref[...]
    proto = lax.dot_general(bb, z, (((1,), (0,)), ((), ())),
                            preferred_element_type=jnp.float32) / n_pix
    proto = _unit_rows(proto)
    zn = _unit_rows(z)
    sim = lax.dot_general(proto, zn, (((1,), (1,)), ((), ())),
                          preferred_element_type=jnp.float32)
    smin = jnp.min(sim, axis=-1, keepdims=True)
    smax = jnp.max(sim, axis=-1, keepdims=True)
    mask_ref[0] = 1.0 - (sim - smin) / jnp.maximum(smax - smin, 1e-12)

    protos = _unit_rows(p0_ref[0])
    row = lax.broadcasted_iota(jnp.int32, (k, n), 0)
    q = jnp.zeros((k, n), jnp.float32)
    idx = jnp.zeros((1, n), jnp.int32)
    for _ in range(n_iters):
        s = lax.dot_general(protos, zn, (((1,), (1,)), ((), ())),
                            preferred_element_type=jnp.float32)
        smx = jnp.max(s, axis=0, keepdims=True)
        e = jnp.exp(s - smx)
        s = e * pl.reciprocal(jnp.sum(e, axis=0, keepdims=True), approx=True)
        p = jnp.exp(s * inv_eps)
        for _ in range(sk_iters):
            p = p * pl.reciprocal(jnp.sum(p, axis=1, keepdims=True),
                                  approx=True)
            p = p * pl.reciprocal(jnp.sum(p, axis=0, keepdims=True),
                                  approx=True)
        q = p
        qmax = jnp.max(q, axis=0, keepdims=True)
        idx = jnp.min(jnp.where(q >= qmax, row, k), axis=0, keepdims=True)
        one_hot = (row == idx).astype(jnp.float32)
        counts = jnp.sum(one_hot, axis=1, keepdims=True)
        new_p = lax.dot_general(one_hot, z, (((1,), (0,)), ((), ())),
                                preferred_element_type=jnp.float32)
        new_p = new_p / jnp.maximum(counts, 1.0)
        protos = _unit_rows(new_p)
    logits_ref[0] = q
    idx_ref[0] = idx


def _wcat(w, dtype):
    nine, cin, cout = w.shape
    assert nine == 9
    w = w.reshape(3, 3, cin, cout).transpose(0, 2, 1, 3)
    return w.reshape(3, cin, 3 * cout).astype(dtype)


def kernel(x, enc1_w, enc1_b, enc2_w, enc2_b, dec1_w, dec1_b,
           dec2_w, dec2_b, proto_key_data):
    b, cin, hh, ww = x.shape
    n = hh * ww
    hid = enc1_w.shape[-1]
    nz = enc2_w.shape[-1]
    k = _N_CLUSTERS

    w1 = _wcat(enc1_w, jnp.float32).reshape(3 * cin, 3 * hid)
    w2 = _wcat(enc2_w, jnp.float32)[::-1].reshape(3 * hid, 3 * nz)
    w3 = _wcat(dec1_w, jnp.bfloat16)[::-1].reshape(3 * nz, 3 * hid)
    w4 = _wcat(dec2_w, jnp.bfloat16)[::-1].reshape(3 * hid, 3 * cin)

    inner = jnp.zeros((hh - 2 * _DUR, ww - 2 * _DUR), jnp.float32)
    bb = jnp.pad(inner, ((_DUR, _DUR), (_DUR, _DUR)),
                 constant_values=1.0).reshape(1, n)
    n_pix = float(hh * ww - (hh - 2 * _DUR) * (ww - 2 * _DUR))

    proto0 = jax.random.normal(jax.random.wrap_key_data(proto_key_data),
                               (b, k, nz), jnp.float32)

    xbar_t, znc, mask_ln, logits_kn, idx_ln = pl.pallas_call(
        functools.partial(_fused_body, hh=hh, ww=ww, n_pix=n_pix,
                          n_iters=3, sk_iters=3, inv_eps=20.0),
        out_shape=(
            jax.ShapeDtypeStruct((b, cin, n), jnp.float32),
            jax.ShapeDtypeStruct((b, nz, n), jnp.float32),
            jax.ShapeDtypeStruct((b, 1, n), jnp.float32),
            jax.ShapeDtypeStruct((b, k, n), jnp.float32),
            jax.ShapeDtypeStruct((b, 1, n), jnp.int32),
        ),
        grid=(b,),
        in_specs=[
            pl.BlockSpec((1, cin, hh, ww), lambda i: (i, 0, 0, 0)),
            pl.BlockSpec((3 * cin, 3 * hid), lambda i: (0, 0)),
            pl.BlockSpec((1, hid), lambda i: (0, 0)),
            pl.BlockSpec((3 * hid, 3 * nz), lambda i: (0, 0)),
            pl.BlockSpec((1, nz), lambda i: (0, 0)),
            pl.BlockSpec((3 * nz, 3 * hid), lambda i: (0, 0)),
            pl.BlockSpec((1, hid), lambda i: (0, 0)),
            pl.BlockSpec((3 * hid, 3 * cin), lambda i: (0, 0)),
            pl.BlockSpec((1, cin), lambda i: (0, 0)),
            pl.BlockSpec((1, n), lambda i: (0, 0)),
            pl.BlockSpec((1, k, nz), lambda i: (i, 0, 0)),
        ],
        out_specs=(
            pl.BlockSpec((1, cin, n), lambda i: (i, 0, 0)),
            pl.BlockSpec((1, nz, n), lambda i: (i, 0, 0)),
            pl.BlockSpec((1, 1, n), lambda i: (i, 0, 0)),
            pl.BlockSpec((1, k, n), lambda i: (i, 0, 0)),
            pl.BlockSpec((1, 1, n), lambda i: (i, 0, 0)),
        ),
        scratch_shapes=[
            pltpu.VMEM((hh + 2, ww, 3 * hid), jnp.float32),
            pltpu.VMEM((hh + 2, ww, 3 * nz), jnp.bfloat16),
            pltpu.VMEM((hh + 2, ww, 3 * hid), jnp.bfloat16),
        ],
        compiler_params=pltpu.CompilerParams(
            dimension_semantics=("parallel",)),
    )(x, w1, enc1_b, w2, enc2_b, w3, dec1_b, w4, dec2_b, bb, proto0)

    x_bar = xbar_t.reshape(b, cin, hh, ww)
    z_nchw = znc.reshape(b, nz, hh, ww)
    mask = mask_ln.reshape(b, n, 1)
    logits = logits_kn.reshape(b, k, hh, ww)
    indexes = idx_ln.reshape(b, hh, ww)
    return x_bar, z_nchw, mask, logits, indexes
